# baseline (speedup 1.0000x reference)
import functools

import numpy as np
import jax
import jax.numpy as jnp
from jax import lax
from jax.experimental import pallas as pl
from jax.experimental.pallas import tpu as pltpu
from jax.experimental.pallas import tpu_sc as plsc

D_MODEL = 64
PADDED_D = 128
SEQ = 200
HALF_SEQ = SEQ // 2
SCALE = 8.0
PE_LEN = SEQ * D_MODEL

_info = plsc.get_sparse_core_info()
_NC, _NS, _L = _info.num_cores, _info.num_subcores, _info.num_lanes
_NW = _NC * _NS

BATCHES_PER_CHUNK = 2
CHUNK = BATCHES_PER_CHUNK * SEQ
PACKED = CHUNK // 2
D_VREGS = D_MODEL // 16


def _pe_bits():
    pos = np.expand_dims(np.arange(0, SEQ), axis=1)
    div_term = np.array(
        [[1 / np.power(10000, 2 * (i // 2) / D_MODEL) for i in range(D_MODEL)]]
    )
    p = pos * div_term
    pe = np.zeros((SEQ, D_MODEL), dtype=np.float32)
    pe[:, 0::2] = np.sin(p[:, 0::2])
    pe[:, 1::2] = np.cos(p[:, 0::2])
    return pe.reshape(-1).view(np.int32)


_PE_BITS = _pe_bits()


@functools.partial(jax.jit, static_argnames=("batch",))
def _embed(xaug, emb, batch):
    n_rows = batch * SEQ
    half = n_rows // 2
    rows_per_w = n_rows // _NW
    half_per_w = rows_per_w // 2
    n_chunks = rows_per_w // CHUNK
    mesh = plsc.VectorSubcoreMesh(core_axis_name="c", subcore_axis_name="s")

    @functools.partial(
        pl.kernel,
        mesh=mesh,
        out_type=jax.ShapeDtypeStruct((half, PADDED_D), jnp.float32),
        scratch_types=[
            pltpu.VMEM((rows_per_w,), jnp.int32),
            pltpu.VMEM((PACKED, D_MODEL), jnp.float32),
            pltpu.VMEM((PACKED, D_MODEL), jnp.float32),
            pltpu.VMEM((PACKED, D_MODEL), jnp.float32),
            pltpu.VMEM((PACKED, D_MODEL), jnp.float32),
            pltpu.VMEM((PACKED, PADDED_D), jnp.float32),
            pltpu.VMEM((PACKED, PADDED_D), jnp.float32),
            pltpu.VMEM((PE_LEN,), jnp.int32),
            pltpu.SemaphoreType.DMA,
            pltpu.SemaphoreType.DMA,
        ],
        compiler_params=pltpu.CompilerParams(
            use_tc_tiling_on_sc=False, needs_layout_passes=False
        ),
    )
    def k(
        xaug_hbm, emb_hbm, out_hbm,
        idx_v, ebuf0, obuf0, ebuf1, obuf1, sbuf0, sbuf1, pe_v, gsem, ssem,
    ):
        wid = lax.axis_index("s") * _NC + lax.axis_index("c")
        ebase = wid * half_per_w
        pltpu.sync_copy(
            xaug_hbm.at[pl.ds(ebase, half_per_w)], idx_v.at[pl.ds(0, half_per_w)]
        )
        pltpu.sync_copy(
            xaug_hbm.at[pl.ds(half + ebase, half_per_w)],
            idx_v.at[pl.ds(half_per_w, half_per_w)],
        )
        pltpu.sync_copy(xaug_hbm.at[pl.ds(n_rows, PE_LEN)], pe_v)
        ebufs = (ebuf0, ebuf1)
        obufs = (obuf0, obuf1)
        sbufs = (sbuf0, sbuf1)

        def gather(c):
            ge = pltpu.async_copy(
                emb_hbm.at[idx_v.at[pl.ds(c * PACKED, PACKED)]], ebufs[c % 2], gsem
            )
            go = pltpu.async_copy(
                emb_hbm.at[idx_v.at[pl.ds(half_per_w + c * PACKED, PACKED)]],
                obufs[c % 2],
                gsem,
            )
            return (ge, go)

        def store(c):
            return pltpu.async_copy(
                sbufs[c % 2],
                out_hbm.at[pl.ds(wid * half_per_w + c * PACKED, PACKED)],
                ssem,
            )

        def compute(c):
            ebuf, obuf, sbuf = ebufs[c % 2], obufs[c % 2], sbufs[c % 2]

            def row_body(j, carry):
                for d in range(2 * D_VREGS):
                    o = j * PADDED_D + d * 16
                    pe_vec = plsc.bitcast(pe_v[pl.ds(o, 16)], jnp.float32)
                    for rep in range(BATCHES_PER_CHUNK):
                        row = rep * HALF_SEQ + j
                        if d < D_VREGS:
                            src = ebuf[row, pl.ds(d * 16, 16)]
                        else:
                            src = obuf[row, pl.ds((d - D_VREGS) * 16, 16)]
                        sbuf[row, pl.ds(d * 16, 16)] = src * SCALE + pe_vec
                return carry

            lax.fori_loop(0, HALF_SEQ, row_body, 0)

        gathers = {0: gather(0)}
        stores = {}
        for c in range(n_chunks):
            for g in gathers[c]:
                g.wait()
            if c + 1 < n_chunks:
                if c >= 1:
                    stores[c - 1].wait()
                gathers[c + 1] = gather(c + 1)
            compute(c)
            stores[c] = store(c)
        stores[n_chunks - 2].wait()
        stores[n_chunks - 1].wait()

    return k(xaug, emb)


def kernel(x, emb):
    b, s = x.shape
    xf = x.reshape(-1)
    xaug = jnp.concatenate([xf[0::2], xf[1::2], jnp.asarray(_PE_BITS)])
    packed = _embed(xaug, emb, b)
    return packed.reshape(b, s, D_MODEL)


# speedup vs baseline: 2.3077x; 2.3077x over previous
import functools

import numpy as np
import jax
import jax.numpy as jnp
from jax import lax
from jax.experimental import pallas as pl
from jax.experimental.pallas import tpu as pltpu
from jax.experimental.pallas import tpu_sc as plsc

D_MODEL = 64
PADDED_D = 128
SEQ = 200
SCALE = 8.0
PE_LEN = SEQ * D_MODEL

_info = plsc.get_sparse_core_info()
_NC, _NS, _L = _info.num_cores, _info.num_subcores, _info.num_lanes
_NW = _NC * _NS

BATCHES_PER_CHUNK = 4
CHUNK = BATCHES_PER_CHUNK * SEQ
D_VREGS = D_MODEL // 16


def _pe_bits():
    pos = np.expand_dims(np.arange(0, SEQ), axis=1)
    div_term = np.array(
        [[1 / np.power(10000, 2 * (i // 2) / D_MODEL) for i in range(D_MODEL)]]
    )
    p = pos * div_term
    pe = np.zeros((SEQ, D_MODEL), dtype=np.float32)
    pe[:, 0::2] = np.sin(p[:, 0::2])
    pe[:, 1::2] = np.cos(p[:, 0::2])
    bits = np.zeros((13312,), dtype=np.int32)
    bits[:PE_LEN] = pe.reshape(-1).view(np.int32)
    return bits


_PE_BITS = _pe_bits()


@functools.partial(jax.jit, static_argnames=("batch",))
def _embed(xaug, emb, batch):
    n_rows = batch * SEQ
    rows_per_w = n_rows // _NW
    n_chunks = rows_per_w // CHUNK
    mesh = plsc.VectorSubcoreMesh(core_axis_name="c", subcore_axis_name="s")

    @functools.partial(
        pl.kernel,
        mesh=mesh,
        out_type=jax.ShapeDtypeStruct((n_rows, PADDED_D), jnp.float32),
        scratch_types=[
            pltpu.VMEM((rows_per_w,), jnp.int32),
            pltpu.VMEM((CHUNK, D_MODEL), jnp.float32),
            pltpu.VMEM((CHUNK, D_MODEL), jnp.float32),
            pltpu.VMEM((PE_LEN,), jnp.int32),
            pltpu.SemaphoreType.DMA,
            pltpu.SemaphoreType.DMA,
        ],
        compiler_params=pltpu.CompilerParams(
            use_tc_tiling_on_sc=False, needs_layout_passes=False
        ),
    )
    def k(xaug_hbm, emb_hbm, out_hbm, idx_v, rows0, rows1, pe_v, gsem, ssem):
        wid = lax.axis_index("s") * _NC + lax.axis_index("c")
        base = wid * rows_per_w
        pltpu.sync_copy(xaug_hbm.at[pl.ds(base, rows_per_w)], idx_v)
        pltpu.sync_copy(xaug_hbm.at[pl.ds(n_rows, PE_LEN)], pe_v)
        bufs = (rows0, rows1)

        def gather(c):
            return pltpu.async_copy(
                emb_hbm.at[idx_v.at[pl.ds(c * CHUNK, CHUNK)]], bufs[c % 2], gsem
            )

        def store(c):
            return pltpu.async_copy(
                bufs[c % 2],
                out_hbm.at[pl.ds(base + c * CHUNK, CHUNK), pl.ds(0, D_MODEL)],
                ssem,
            )

        def compute(buf):
            def row_body(r, carry):
                for d in range(D_VREGS):
                    o = (r * D_VREGS + d) * 16
                    pe_vec = plsc.bitcast(pe_v[pl.ds(o, 16)], jnp.float32)
                    for rep in range(BATCHES_PER_CHUNK):
                        row = rep * SEQ + r
                        sl = pl.ds(d * 16, 16)
                        buf[row, sl] = buf[row, sl] * SCALE + pe_vec
                return carry

            lax.fori_loop(0, SEQ, row_body, 0)

        gathers = {0: gather(0)}
        stores = {}
        for c in range(n_chunks):
            gathers[c].wait()
            if c + 1 < n_chunks:
                if c >= 1:
                    stores[c - 1].wait()
                gathers[c + 1] = gather(c + 1)
            compute(bufs[c % 2])
            stores[c] = store(c)
        stores[n_chunks - 2].wait()
        stores[n_chunks - 1].wait()

    return k(xaug, emb)


def kernel(x, emb):
    b, s = x.shape
    xaug = jnp.concatenate([x.reshape(-1), jnp.asarray(_PE_BITS)])
    out128 = _embed(xaug, emb, b)
    return out128[:, :D_MODEL].reshape(b, s, D_MODEL)
